# baseline (speedup 1.0000x reference)
import functools

import jax
import jax.numpy as jnp
from jax.experimental import pallas as pl
from jax.experimental.pallas import tpu as pltpu

CODEBOOK_SIZE = 1024
LATENT_DIM = 64
RQ_LEVELS = 4
BLOCK_N = 1024


def _level_kernel(last, r_ref, rsq_ref, qsum_ref, z_ref, cb_ref, cbsq_ref,
                  cbhi_ref, cbmid_ref, cblo_ref,
                  rout_ref, qsout_ref, idx_ref, usage_ref, loss_ref):
    step = pl.program_id(0)

    @pl.when(step == 0)
    def _init():
        usage_ref[...] = jnp.zeros_like(usage_ref)
        loss_ref[...] = jnp.zeros_like(loss_ref)

    r = r_ref[...]
    cb = cb_ref[...]
    cross = jax.lax.dot_general(
        r, cb, (((1,), (1,)), ((), ())),
        preferred_element_type=jnp.float32)
    d = rsq_ref[...] - 2.0 * cross + cbsq_ref[...]
    dmin = jnp.min(d, axis=1, keepdims=True)
    col_iota = jax.lax.broadcasted_iota(
        jnp.int32, (BLOCK_N, CODEBOOK_SIZE), 1)
    idx = jnp.min(jnp.where(d == dmin, col_iota, CODEBOOK_SIZE),
                  axis=1, keepdims=True)
    onehot = (col_iota == idx).astype(jnp.float32)
    gdot = lambda ref: jax.lax.dot_general(
        onehot, ref[...], (((1,), (0,)), ((), ())),
        preferred_element_type=jnp.float32)
    q = (gdot(cbhi_ref) + gdot(cbmid_ref)) + gdot(cblo_ref)

    diff = r - q
    rout_ref[...] = diff
    qs = qsum_ref[...] + q
    if last:
        z = z_ref[...]
        qsout_ref[...] = z + (qs - z)
    else:
        qsout_ref[...] = qs
    idx_ref[...] = idx
    usage_ref[...] += jnp.sum(onehot, axis=0, keepdims=True)
    loss_ref[...] += jnp.reshape(jnp.sum(diff * diff), (1, 1))


def _make_level(n, last):
    row = lambda i: (i, 0)
    rep = lambda i: (0, 0)
    return pl.pallas_call(
        functools.partial(_level_kernel, last),
        grid=(n // BLOCK_N,),
        in_specs=[
            pl.BlockSpec((BLOCK_N, LATENT_DIM), row),
            pl.BlockSpec((BLOCK_N, 1), row),
            pl.BlockSpec((BLOCK_N, LATENT_DIM), row),
            pl.BlockSpec((BLOCK_N, LATENT_DIM), row),
            pl.BlockSpec((CODEBOOK_SIZE, LATENT_DIM), rep),
            pl.BlockSpec((1, CODEBOOK_SIZE), rep),
            pl.BlockSpec((CODEBOOK_SIZE, LATENT_DIM), rep),
            pl.BlockSpec((CODEBOOK_SIZE, LATENT_DIM), rep),
            pl.BlockSpec((CODEBOOK_SIZE, LATENT_DIM), rep),
        ],
        out_specs=[
            pl.BlockSpec((BLOCK_N, LATENT_DIM), row),
            pl.BlockSpec((BLOCK_N, LATENT_DIM), row),
            pl.BlockSpec((BLOCK_N, 1), row),
            pl.BlockSpec((1, CODEBOOK_SIZE), rep),
            pl.BlockSpec((1, 1), rep),
        ],
        out_shape=[
            jax.ShapeDtypeStruct((n, LATENT_DIM), jnp.float32),
            jax.ShapeDtypeStruct((n, LATENT_DIM), jnp.float32),
            jax.ShapeDtypeStruct((n, 1), jnp.int32),
            jax.ShapeDtypeStruct((1, CODEBOOK_SIZE), jnp.float32),
            jax.ShapeDtypeStruct((1, 1), jnp.float32),
        ],
        compiler_params=pltpu.CompilerParams(
            dimension_semantics=("parallel",)),
    )


@jax.jit
def kernel(z, codebook):
    n = z.shape[0]
    cbsq = (codebook ** 2).sum(axis=1)[None, :]
    def trunc16(x):
        u = jax.lax.bitcast_convert_type(x, jnp.uint32)
        return jax.lax.bitcast_convert_type(
            u & jnp.uint32(0xFFFF0000), jnp.float32)
    cb_hi = trunc16(codebook)
    rem = codebook - cb_hi
    cb_mid = trunc16(rem)
    cb_lo = rem - cb_mid
    inv_count = 1.0 / (n * LATENT_DIM)

    residual = z
    qsum = jnp.zeros_like(z)
    loss = jnp.float32(0.0)
    usage = jnp.zeros((1, CODEBOOK_SIZE), jnp.float32)
    codes = []
    for lvl in range(RQ_LEVELS):
        rsq = (residual ** 2).sum(axis=1, keepdims=True)
        residual, qsum, idx, u, s = _make_level(n, lvl == RQ_LEVELS - 1)(
            residual, rsq, qsum, z, codebook, cbsq, cb_hi, cb_mid, cb_lo)
        codes.append(idx[:, 0])
        usage = usage + u
        loss = loss + s[0, 0] * inv_count

    return qsum, loss, jnp.stack(codes, axis=1), usage[0]


# speedup vs baseline: 1.2093x; 1.2093x over previous
import functools

import jax
import jax.numpy as jnp
from jax.experimental import pallas as pl
from jax.experimental.pallas import tpu as pltpu

CODEBOOK_SIZE = 1024
LATENT_DIM = 64
RQ_LEVELS = 4
BLOCK_N = 1024


def _level_kernel(last, r_ref, rsq_ref, qsum_ref, z_ref, cb_ref, cbsq_ref,
                  cbparts_ref,
                  rout_ref, qsout_ref, idx_ref, usage_ref, loss_ref):
    step = pl.program_id(0)

    @pl.when(step == 0)
    def _init():
        usage_ref[...] = jnp.zeros_like(usage_ref)
        loss_ref[...] = jnp.zeros_like(loss_ref)

    r = r_ref[...]
    cb = cb_ref[...]
    cross = jax.lax.dot_general(
        r, cb, (((1,), (1,)), ((), ())),
        preferred_element_type=jnp.float32)
    d = rsq_ref[...] - 2.0 * cross + cbsq_ref[...]
    dmin = jnp.min(d, axis=1, keepdims=True)
    col_iota = jax.lax.broadcasted_iota(
        jnp.int32, (BLOCK_N, CODEBOOK_SIZE), 1)
    idx = jnp.min(jnp.where(d == dmin, col_iota, CODEBOOK_SIZE),
                  axis=1, keepdims=True)
    onehot = (col_iota == idx).astype(jnp.float32)
    g = jax.lax.dot_general(
        onehot.astype(jnp.bfloat16), cbparts_ref[...],
        (((1,), (0,)), ((), ())),
        preferred_element_type=jnp.float32)
    q = (g[:, 0:LATENT_DIM] + g[:, LATENT_DIM:2 * LATENT_DIM]) \
        + g[:, 2 * LATENT_DIM:3 * LATENT_DIM]

    diff = r - q
    rout_ref[...] = diff
    qs = qsum_ref[...] + q
    if last:
        z = z_ref[...]
        qsout_ref[...] = z + (qs - z)
    else:
        qsout_ref[...] = qs
    idx_ref[...] = idx
    usage_ref[...] += jnp.sum(onehot, axis=0, keepdims=True)
    loss_ref[...] += jnp.reshape(jnp.sum(diff * diff), (1, 1))


def _make_level(n, last):
    row = lambda i: (i, 0)
    rep = lambda i: (0, 0)
    return pl.pallas_call(
        functools.partial(_level_kernel, last),
        grid=(n // BLOCK_N,),
        in_specs=[
            pl.BlockSpec((BLOCK_N, LATENT_DIM), row),
            pl.BlockSpec((BLOCK_N, 1), row),
            pl.BlockSpec((BLOCK_N, LATENT_DIM), row),
            pl.BlockSpec((BLOCK_N, LATENT_DIM), row),
            pl.BlockSpec((CODEBOOK_SIZE, LATENT_DIM), rep),
            pl.BlockSpec((1, CODEBOOK_SIZE), rep),
            pl.BlockSpec((CODEBOOK_SIZE, 3 * LATENT_DIM), rep),
        ],
        out_specs=[
            pl.BlockSpec((BLOCK_N, LATENT_DIM), row),
            pl.BlockSpec((BLOCK_N, LATENT_DIM), row),
            pl.BlockSpec((BLOCK_N, 1), row),
            pl.BlockSpec((1, CODEBOOK_SIZE), rep),
            pl.BlockSpec((1, 1), rep),
        ],
        out_shape=[
            jax.ShapeDtypeStruct((n, LATENT_DIM), jnp.float32),
            jax.ShapeDtypeStruct((n, LATENT_DIM), jnp.float32),
            jax.ShapeDtypeStruct((n, 1), jnp.int32),
            jax.ShapeDtypeStruct((1, CODEBOOK_SIZE), jnp.float32),
            jax.ShapeDtypeStruct((1, 1), jnp.float32),
        ],
        compiler_params=pltpu.CompilerParams(
            dimension_semantics=("arbitrary",)),
    )


@jax.jit
def kernel(z, codebook):
    n = z.shape[0]
    cbsq = (codebook ** 2).sum(axis=1)[None, :]
    def trunc16(x):
        u = jax.lax.bitcast_convert_type(x, jnp.uint32)
        return jax.lax.bitcast_convert_type(
            u & jnp.uint32(0xFFFF0000), jnp.float32)
    cb_hi = trunc16(codebook)
    rem = codebook - cb_hi
    cb_mid = trunc16(rem)
    cb_lo = rem - cb_mid
    cb_parts = jnp.concatenate([cb_hi, cb_mid, cb_lo],
                               axis=1).astype(jnp.bfloat16)
    inv_count = 1.0 / (n * LATENT_DIM)

    residual = z
    qsum = jnp.zeros_like(z)
    loss = jnp.float32(0.0)
    usage = jnp.zeros((1, CODEBOOK_SIZE), jnp.float32)
    codes = []
    for lvl in range(RQ_LEVELS):
        rsq = (residual ** 2).sum(axis=1, keepdims=True)
        residual, qsum, idx, u, s = _make_level(n, lvl == RQ_LEVELS - 1)(
            residual, rsq, qsum, z, codebook, cbsq, cb_parts)
        codes.append(idx[:, 0])
        usage = usage + u
        loss = loss + s[0, 0] * inv_count

    return qsum, loss, jnp.stack(codes, axis=1), usage[0]


# speedup vs baseline: 1.2903x; 1.0670x over previous
import functools

import jax
import jax.numpy as jnp
from jax.experimental import pallas as pl
from jax.experimental.pallas import tpu as pltpu

CODEBOOK_SIZE = 1024
LATENT_DIM = 64
RQ_LEVELS = 4
BLOCK_N = 1024


def _level_kernel(last, *refs):
    if last:
        (r_ref, rsq_ref, qsum_ref, z_ref, cb_ref, cbsq_ref,
         cbparts_ref) = refs[:7]
        qsout_ref, idx_ref, usage_ref, loss_ref = refs[7:]
        rout_ref = None
    else:
        (r_ref, rsq_ref, qsum_ref, cb_ref, cbsq_ref,
         cbparts_ref) = refs[:6]
        rout_ref, qsout_ref, idx_ref, usage_ref, loss_ref = refs[6:]
        z_ref = None
    step = pl.program_id(0)

    @pl.when(step == 0)
    def _init():
        usage_ref[...] = jnp.zeros_like(usage_ref)
        loss_ref[...] = jnp.zeros_like(loss_ref)

    r = r_ref[...]
    cb = cb_ref[...]
    cross = jax.lax.dot_general(
        r, cb, (((1,), (1,)), ((), ())),
        preferred_element_type=jnp.float32)
    d = rsq_ref[...] - 2.0 * cross + cbsq_ref[...]
    dmin = jnp.min(d, axis=1, keepdims=True)
    col_iota = jax.lax.broadcasted_iota(
        jnp.int32, (BLOCK_N, CODEBOOK_SIZE), 1)
    idx = jnp.min(jnp.where(d == dmin, col_iota, CODEBOOK_SIZE),
                  axis=1, keepdims=True)
    onehot = (col_iota == idx).astype(jnp.bfloat16)
    g = jax.lax.dot_general(
        onehot, cbparts_ref[...],
        (((1,), (0,)), ((), ())),
        preferred_element_type=jnp.float32)
    q = (g[:, 0:LATENT_DIM] + g[:, LATENT_DIM:2 * LATENT_DIM]) \
        + g[:, 2 * LATENT_DIM:3 * LATENT_DIM]

    diff = r - q
    qs = qsum_ref[...] + q
    if last:
        z = z_ref[...]
        qsout_ref[...] = z + (qs - z)
    else:
        rout_ref[...] = diff
        qsout_ref[...] = qs
    idx_ref[...] = idx
    ones_row = jnp.ones((1, BLOCK_N), jnp.bfloat16)
    usage_ref[...] += jax.lax.dot_general(
        ones_row, onehot, (((1,), (0,)), ((), ())),
        preferred_element_type=jnp.float32)
    loss_ref[...] += jnp.reshape(jnp.sum(diff * diff), (1, 1))


def _make_level(n, last):
    row = lambda i: (i, 0)
    rep = lambda i: (0, 0)
    out_specs = [
        pl.BlockSpec((BLOCK_N, LATENT_DIM), row),
        pl.BlockSpec((BLOCK_N, 1), row),
        pl.BlockSpec((1, CODEBOOK_SIZE), rep),
        pl.BlockSpec((1, 1), rep),
    ]
    out_shape = [
        jax.ShapeDtypeStruct((n, LATENT_DIM), jnp.float32),
        jax.ShapeDtypeStruct((n, 1), jnp.int32),
        jax.ShapeDtypeStruct((1, CODEBOOK_SIZE), jnp.float32),
        jax.ShapeDtypeStruct((1, 1), jnp.float32),
    ]
    if not last:
        out_specs.insert(0, pl.BlockSpec((BLOCK_N, LATENT_DIM), row))
        out_shape.insert(0, jax.ShapeDtypeStruct((n, LATENT_DIM), jnp.float32))
    return pl.pallas_call(
        functools.partial(_level_kernel, last),
        grid=(n // BLOCK_N,),
        in_specs=(
            [pl.BlockSpec((BLOCK_N, LATENT_DIM), row),
             pl.BlockSpec((BLOCK_N, 1), row),
             pl.BlockSpec((BLOCK_N, LATENT_DIM), row)]
            + ([pl.BlockSpec((BLOCK_N, LATENT_DIM), row)] if last else [])
            + [pl.BlockSpec((CODEBOOK_SIZE, LATENT_DIM), rep),
               pl.BlockSpec((1, CODEBOOK_SIZE), rep),
               pl.BlockSpec((CODEBOOK_SIZE, 3 * LATENT_DIM), rep)]
        ),
        out_specs=out_specs,
        out_shape=out_shape,
        compiler_params=pltpu.CompilerParams(
            dimension_semantics=("arbitrary",)),
    )


@jax.jit
def kernel(z, codebook):
    n = z.shape[0]
    cbsq = (codebook ** 2).sum(axis=1)[None, :]
    def trunc16(x):
        u = jax.lax.bitcast_convert_type(x, jnp.uint32)
        return jax.lax.bitcast_convert_type(
            u & jnp.uint32(0xFFFF0000), jnp.float32)
    cb_hi = trunc16(codebook)
    rem = codebook - cb_hi
    cb_mid = trunc16(rem)
    cb_lo = rem - cb_mid
    cb_parts = jnp.concatenate([cb_hi, cb_mid, cb_lo],
                               axis=1).astype(jnp.bfloat16)
    inv_count = 1.0 / (n * LATENT_DIM)

    residual = z
    qsum = jnp.zeros_like(z)
    loss = jnp.float32(0.0)
    usage = jnp.zeros((1, CODEBOOK_SIZE), jnp.float32)
    codes = []
    for lvl in range(RQ_LEVELS):
        last = lvl == RQ_LEVELS - 1
        rsq = (residual ** 2).sum(axis=1, keepdims=True)
        ins = ((residual, rsq, qsum, z) if last
               else (residual, rsq, qsum))
        outs = _make_level(n, last)(*ins, codebook, cbsq, cb_parts)
        if last:
            qsum, idx, u, s = outs
        else:
            residual, qsum, idx, u, s = outs
        codes.append(idx[:, 0])
        usage = usage + u
        loss = loss + s[0, 0] * inv_count

    return qsum, loss, jnp.stack(codes, axis=1), usage[0]
